# Initial kernel scaffold; baseline (speedup 1.0000x reference)
import functools

import jax
import jax.numpy as jnp
from jax import lax
from jax.experimental import pallas as pl
from jax.experimental.pallas import tpu as pltpu
from jax.experimental.pallas import tpu_sc as plsc

VOCAB = 100000
EMB = 64
B = 4096
L = 200
N_CLASSES = 20
DP = 32
NC = 2
NS = 16
NW = NC * NS
SEQ_PER_W = B // NW
TOK_PER_W = SEQ_PER_W * L
TOK_STEP = 128
N_STEPS = TOK_PER_W // TOK_STEP


_ROWS_BLK = 2000


def _proj_body(tbl_ref, wt_ref, out_ref):
    i = pl.program_id(0)
    rows = lax.broadcasted_iota(jnp.int32, (_ROWS_BLK, 1), 0) + i * _ROWS_BLK
    tbl = jnp.where(rows == 0, 0.0, tbl_ref[...])
    out_ref[...] = jnp.dot(tbl, wt_ref[...], preferred_element_type=jnp.float32)


def _compute_proj(table, wt):
    return pl.pallas_call(
        _proj_body,
        grid=(VOCAB // _ROWS_BLK,),
        in_specs=[
            pl.BlockSpec((_ROWS_BLK, EMB), lambda i: (i, 0)),
            pl.BlockSpec((EMB, DP), lambda i: (0, 0)),
        ],
        out_specs=pl.BlockSpec((_ROWS_BLK, DP), lambda i: (i, 0)),
        out_shape=jax.ShapeDtypeStruct((VOCAB, DP), jnp.float32),
    )(table, wt)


_sc_mesh = plsc.VectorSubcoreMesh(core_axis_name="c", subcore_axis_name="s")


@functools.partial(
    pl.kernel,
    out_type=jax.ShapeDtypeStruct((B, DP), jnp.float32),
    mesh=_sc_mesh,
    scratch_types=[
        pltpu.VMEM((N_STEPS, TOK_STEP), jnp.int32),
        pltpu.VMEM((N_STEPS, TOK_STEP), jnp.int32),
        pltpu.VMEM((TOK_STEP, DP), jnp.float32),
        pltpu.VMEM((SEQ_PER_W, DP), jnp.float32),
        pltpu.VMEM_SHARED((B, DP), jnp.float32),
        pltpu.SemaphoreType.DMA,
    ],
)
def _sc_gather_sum(proj_hbm, tok_hbm, acc_hbm, tok_v, pat_v, rows_v, stage_v,
                   acc_sh, sem):
    c = lax.axis_index("c")
    s = lax.axis_index("s")
    wid = c * NS + s
    seq_base = wid * SEQ_PER_W

    pltpu.sync_copy(tok_hbm.at[pl.ds(wid * N_STEPS, N_STEPS)], tok_v)

    lane = lax.broadcasted_iota(jnp.int32, (16,), 0)

    def _pat_body(g, carry):
        t0 = g * TOK_STEP
        for j in range(TOK_STEP // 16):
            t = t0 + j * 16 + lane
            pat_v[g, pl.ds(j * 16, 16)] = seq_base + t // L
        return carry

    lax.fori_loop(0, N_STEPS, _pat_body, 0)

    zero16 = jnp.zeros((16,), jnp.float32)

    def _zero_body(i, carry):
        stage_v[i, pl.ds(0, 16)] = zero16
        stage_v[i, pl.ds(16, 16)] = zero16
        return carry

    lax.fori_loop(0, SEQ_PER_W, _zero_body, 0)
    pltpu.sync_copy(stage_v, acc_sh.at[pl.ds(seq_base, SEQ_PER_W)])

    def _step(g, carry):
        pltpu.async_copy(proj_hbm.at[tok_v.at[g]], rows_v, sem).wait()
        pltpu.sync_copy(rows_v, acc_sh.at[pat_v.at[g]], add=True)
        return carry

    lax.fori_loop(0, N_STEPS, _step, 0)

    pltpu.sync_copy(acc_sh.at[pl.ds(seq_base, SEQ_PER_W)], stage_v)
    pltpu.sync_copy(stage_v, acc_hbm.at[pl.ds(seq_base, SEQ_PER_W)])


def _fin_body(acc_ref, len_ref, b_ref, out_ref):
    out_ref[...] = acc_ref[...][:, :N_CLASSES] / len_ref[...] + b_ref[...]


def _finalize(acc, length_f32, bias):
    return pl.pallas_call(
        _fin_body,
        out_shape=jax.ShapeDtypeStruct((B, N_CLASSES), jnp.float32),
    )(acc, length_f32, bias)


def kernel(data, length, table, W, b):
    wt = jnp.zeros((EMB, DP), jnp.float32).at[:, :N_CLASSES].set(W.T)
    proj = _compute_proj(table, wt)
    tok = data.astype(jnp.int32).reshape(B * L // TOK_STEP, TOK_STEP)
    acc = _sc_gather_sum(proj, tok)
    out = _finalize(acc, length.astype(jnp.float32).reshape(B, 1),
                    b.reshape(1, N_CLASSES))
    return out


# speedup vs baseline: 7.1888x; 7.1888x over previous
import functools

import jax
import jax.numpy as jnp
import numpy as np
from jax import lax
from jax.experimental import pallas as pl
from jax.experimental.pallas import tpu as pltpu
from jax.experimental.pallas import tpu_sc as plsc

VOCAB = 100000
EMB = 64
B = 4096
L = 200
N_CLASSES = 20
DP = 32
NC = 2
NS = 16
NW = NC * NS
SEQ_PER_W = B // NW
TOK_PER_W = SEQ_PER_W * L
TOK_STEP = 128
N_STEPS = TOK_PER_W // TOK_STEP


_ROWS_BLK = 2000


def _proj_body(tbl_ref, wt_ref, out_ref):
    i = pl.program_id(0)
    rows = lax.broadcasted_iota(jnp.int32, (_ROWS_BLK, 1), 0) + i * _ROWS_BLK
    tbl = jnp.where(rows == 0, 0.0, tbl_ref[...])
    out_ref[...] = jnp.dot(tbl, wt_ref[...], preferred_element_type=jnp.float32)


def _compute_proj(table, wt):
    return pl.pallas_call(
        _proj_body,
        grid=(VOCAB // _ROWS_BLK,),
        in_specs=[
            pl.BlockSpec((_ROWS_BLK, EMB), lambda i: (i, 0)),
            pl.BlockSpec((EMB, DP), lambda i: (0, 0)),
        ],
        out_specs=pl.BlockSpec((_ROWS_BLK, DP), lambda i: (i, 0)),
        out_shape=jax.ShapeDtypeStruct((VOCAB, DP), jnp.float32),
    )(table, wt)


_sc_mesh = plsc.VectorSubcoreMesh(core_axis_name="c", subcore_axis_name="s",
                                  num_cores=NC, num_subcores=NS)


_PAT = (np.arange(B * L, dtype=np.int64) // L).astype(np.int32).reshape(
    B * L // TOK_STEP, TOK_STEP)


@functools.partial(
    pl.kernel,
    out_type=jax.ShapeDtypeStruct((B, DP), jnp.float32),
    mesh=_sc_mesh,
    scratch_types=[
        pltpu.VMEM((N_STEPS, TOK_STEP), jnp.int32),
        pltpu.VMEM((N_STEPS, TOK_STEP), jnp.int32),
        pltpu.VMEM((TOK_STEP, DP), jnp.float32),
        pltpu.VMEM((SEQ_PER_W, DP), jnp.float32),
        pltpu.VMEM_SHARED((B, DP), jnp.float32),
        pltpu.SemaphoreType.DMA,
    ],
    compiler_params=pltpu.CompilerParams(use_tc_tiling_on_sc=False),
)
def _sc_gather_sum(proj_hbm, tok_hbm, pat_hbm, zeros_hbm, acc_hbm,
                   tok_v, pat_v, rows_v, stage_v, acc_sh, sem):
    c = lax.axis_index("c")
    s = lax.axis_index("s")
    wid = c * NS + s
    seq_base = wid * SEQ_PER_W
    row_base = wid * N_STEPS

    pltpu.sync_copy(tok_hbm.at[pl.ds(row_base, N_STEPS)], tok_v)
    pltpu.sync_copy(pat_hbm.at[pl.ds(row_base, N_STEPS)], pat_v)

    pltpu.sync_copy(zeros_hbm, stage_v)
    pltpu.sync_copy(stage_v, acc_sh.at[pl.ds(seq_base, SEQ_PER_W)])

    def _step(g, carry):
        pltpu.async_copy(proj_hbm.at[tok_v.at[g]], rows_v, sem).wait()
        pltpu.sync_copy(rows_v, acc_sh.at[pat_v.at[g]], add=True)
        return carry

    lax.fori_loop(0, N_STEPS, _step, 0)

    pltpu.sync_copy(acc_sh.at[pl.ds(seq_base, SEQ_PER_W)], stage_v)
    pltpu.sync_copy(stage_v, acc_hbm.at[pl.ds(seq_base, SEQ_PER_W)])


def _fin_body(acc_ref, len_ref, b_ref, out_ref):
    out_ref[...] = acc_ref[...][:, :N_CLASSES] / len_ref[...] + b_ref[...]


def _finalize(acc, length_f32, bias):
    return pl.pallas_call(
        _fin_body,
        out_shape=jax.ShapeDtypeStruct((B, N_CLASSES), jnp.float32),
    )(acc, length_f32, bias)


def kernel(data, length, table, W, b):
    wt = jnp.zeros((EMB, DP), jnp.float32).at[:, :N_CLASSES].set(W.T)
    proj = _compute_proj(table, wt)
    tok = data.astype(jnp.int32).reshape(B * L // TOK_STEP, TOK_STEP)
    pat = jnp.asarray(_PAT)
    zeros = jnp.zeros((SEQ_PER_W, DP), jnp.float32)
    acc = _sc_gather_sum(proj, tok, pat, zeros)
    out = _finalize(acc, length.astype(jnp.float32).reshape(B, 1),
                    b.reshape(1, N_CLASSES))
    return out


# speedup vs baseline: 10.1205x; 1.4078x over previous
import functools

import jax
import jax.numpy as jnp
import numpy as np
from jax import lax
from jax.experimental import pallas as pl
from jax.experimental.pallas import tpu as pltpu
from jax.experimental.pallas import tpu_sc as plsc

VOCAB = 100000
EMB = 64
B = 4096
L = 200
N_CLASSES = 20
DP = 32
NC = 2
NS = 16
NW = NC * NS
SEQ_PER_W = B // NW
TOK_PER_W = SEQ_PER_W * L
TOK_STEP = 128
N_STEPS = TOK_PER_W // TOK_STEP
NBUF = 8


_ROWS_BLK = 2000


def _proj_body(tbl_ref, wt_ref, out_ref):
    i = pl.program_id(0)
    rows = lax.broadcasted_iota(jnp.int32, (_ROWS_BLK, 1), 0) + i * _ROWS_BLK
    tbl = jnp.where(rows == 0, 0.0, tbl_ref[...])
    out_ref[...] = jnp.dot(tbl, wt_ref[...], preferred_element_type=jnp.float32)


def _compute_proj(table, wt):
    return pl.pallas_call(
        _proj_body,
        grid=(VOCAB // _ROWS_BLK,),
        in_specs=[
            pl.BlockSpec((_ROWS_BLK, EMB), lambda i: (i, 0)),
            pl.BlockSpec((EMB, DP), lambda i: (0, 0)),
        ],
        out_specs=pl.BlockSpec((_ROWS_BLK, DP), lambda i: (i, 0)),
        out_shape=jax.ShapeDtypeStruct((VOCAB, DP), jnp.float32),
    )(table, wt)


_sc_mesh = plsc.VectorSubcoreMesh(core_axis_name="c", subcore_axis_name="s",
                                  num_cores=NC, num_subcores=NS)


_PAT = (np.arange(B * L, dtype=np.int64) // L).astype(np.int32).reshape(
    B * L // TOK_STEP, TOK_STEP)


@functools.partial(
    pl.kernel,
    out_type=jax.ShapeDtypeStruct((B, DP), jnp.float32),
    mesh=_sc_mesh,
    scratch_types=[
        pltpu.VMEM((N_STEPS, TOK_STEP), jnp.int32),
        pltpu.VMEM((N_STEPS, TOK_STEP), jnp.int32),
        pltpu.VMEM((NBUF, TOK_STEP, DP), jnp.float32),
        pltpu.VMEM((SEQ_PER_W, DP), jnp.float32),
        pltpu.VMEM_SHARED((B, DP), jnp.float32),
        pltpu.SemaphoreType.DMA((NBUF,)),
        pltpu.SemaphoreType.DMA((NBUF,)),
    ],
    compiler_params=pltpu.CompilerParams(use_tc_tiling_on_sc=False),
)
def _sc_gather_sum(proj_hbm, tok_hbm, pat_hbm, zeros_hbm, acc_hbm,
                   tok_v, pat_v, rows_v, stage_v, acc_sh, gsem, ssem):
    c = lax.axis_index("c")
    s = lax.axis_index("s")
    wid = c * NS + s
    seq_base = wid * SEQ_PER_W
    row_base = wid * N_STEPS

    pltpu.sync_copy(tok_hbm.at[pl.ds(row_base, N_STEPS)], tok_v)
    pltpu.sync_copy(pat_hbm.at[pl.ds(row_base, N_STEPS)], pat_v)

    pltpu.sync_copy(zeros_hbm, stage_v)
    pltpu.sync_copy(stage_v, acc_sh.at[pl.ds(seq_base, SEQ_PER_W)])

    def _gather(g, b):
        pltpu.async_copy(proj_hbm.at[tok_v.at[g]], rows_v.at[b], gsem.at[b])

    def _gather_wait(g, b):
        pltpu.make_async_copy(proj_hbm.at[tok_v.at[g]], rows_v.at[b],
                              gsem.at[b]).wait()

    def _scatter(g, b):
        pltpu.async_copy(rows_v.at[b], acc_sh.at[pat_v.at[g]], ssem.at[b],
                         add=True)

    def _scatter_wait(g, b):
        pltpu.make_async_copy(rows_v.at[b], acc_sh.at[pat_v.at[g]],
                              ssem.at[b]).wait()

    for b in range(NBUF):
        _gather(b, b)

    def _step(i, carry):
        base = i * NBUF
        for b in range(NBUF):
            _gather_wait(base + b, b)
            _scatter(base + b, b)
        for b in range(NBUF):
            _scatter_wait(base + b, b)
            _gather(base + NBUF + b, b)
        return carry

    lax.fori_loop(0, N_STEPS // NBUF - 1, _step, 0, unroll=False)

    last = N_STEPS - NBUF
    for b in range(NBUF):
        _gather_wait(last + b, b)
        _scatter(last + b, b)
    for b in range(NBUF):
        _scatter_wait(last + b, b)

    pltpu.sync_copy(acc_sh.at[pl.ds(seq_base, SEQ_PER_W)], stage_v)
    pltpu.sync_copy(stage_v, acc_hbm.at[pl.ds(seq_base, SEQ_PER_W)])


def _fin_body(acc_ref, len_ref, b_ref, out_ref):
    out_ref[...] = acc_ref[...][:, :N_CLASSES] / len_ref[...] + b_ref[...]


def _finalize(acc, length_f32, bias):
    return pl.pallas_call(
        _fin_body,
        out_shape=jax.ShapeDtypeStruct((B, N_CLASSES), jnp.float32),
    )(acc, length_f32, bias)


def kernel(data, length, table, W, b):
    wt = jnp.zeros((EMB, DP), jnp.float32).at[:, :N_CLASSES].set(W.T)
    proj = _compute_proj(table, wt)
    tok = data.astype(jnp.int32).reshape(B * L // TOK_STEP, TOK_STEP)
    pat = jnp.asarray(_PAT)
    zeros = jnp.zeros((SEQ_PER_W, DP), jnp.float32)
    acc = _sc_gather_sum(proj, tok, pat, zeros)
    out = _finalize(acc, length.astype(jnp.float32).reshape(B, 1),
                    b.reshape(1, N_CLASSES))
    return out


# speedup vs baseline: 21.6456x; 2.1388x over previous
import functools

import jax
import jax.numpy as jnp
import numpy as np
from jax import lax
from jax.experimental import pallas as pl
from jax.experimental.pallas import tpu as pltpu
from jax.experimental.pallas import tpu_sc as plsc

VOCAB = 100000
EMB = 64
B = 4096
L = 200
N_CLASSES = 20
DP = 32
PACK = 128 // DP
NC = 2
NS = 16
NW = NC * NS
SEQ_PER_W = B // NW
NBUF = 8


_ROWS_BLK = 4096


_CHUNK = _ROWS_BLK // PACK
_NBLK = (VOCAB + _ROWS_BLK - 1) // _ROWS_BLK
_VPAD = _NBLK * _ROWS_BLK


def _proj_body(tblt_ref, wt_ref, out_ref):
    i = pl.program_id(0)
    p = lax.dot_general(tblt_ref[...], wt_ref[...],
                        (((0,), (0,)), ((), ())),
                        preferred_element_type=jnp.float32)
    rows = lax.broadcasted_iota(jnp.int32, (_ROWS_BLK, 1), 0) + i * _ROWS_BLK
    p = jnp.where(rows == 0, 0.0, p)
    out_ref[...] = jnp.concatenate(
        [p[k * _CHUNK:(k + 1) * _CHUNK, :] for k in range(PACK)], axis=1)


def _compute_proj(tablet, wt):
    return pl.pallas_call(
        _proj_body,
        grid=(_NBLK,),
        in_specs=[
            pl.BlockSpec((EMB, _ROWS_BLK), lambda i: (0, i)),
            pl.BlockSpec((EMB, DP), lambda i: (0, 0)),
        ],
        out_specs=pl.BlockSpec((_CHUNK, 128), lambda i: (i, 0)),
        out_shape=jax.ShapeDtypeStruct((_VPAD // PACK, 128), jnp.float32),
    )(tablet, wt)


_sc_mesh = plsc.VectorSubcoreMesh(core_axis_name="c", subcore_axis_name="s",
                                  num_cores=NC, num_subcores=NS)


@functools.partial(
    pl.kernel,
    out_type=jax.ShapeDtypeStruct((B, DP), jnp.float32),
    mesh=_sc_mesh,
    scratch_types=[
        pltpu.VMEM((L, SEQ_PER_W), jnp.int32),
        pltpu.VMEM((SEQ_PER_W,), jnp.int32),
        pltpu.VMEM((NBUF, SEQ_PER_W, DP), jnp.float32),
        pltpu.VMEM((SEQ_PER_W, DP), jnp.float32),
        pltpu.VMEM_SHARED((B, DP), jnp.float32),
        pltpu.SemaphoreType.DMA((NBUF,)),
        pltpu.SemaphoreType.DMA((NBUF,)),
    ],
    compiler_params=pltpu.CompilerParams(use_tc_tiling_on_sc=False),
)
def _sc_gather_sum(proj_hbm, tokt_hbm, pat_hbm, zeros_hbm, acc_hbm,
                   tok_v, pat_v, rows_v, stage_v, acc_sh, gsem, ssem):
    c = lax.axis_index("c")
    s = lax.axis_index("s")
    wid = c * NS + s
    seq_base = wid * SEQ_PER_W

    pltpu.sync_copy(tokt_hbm.at[:, pl.ds(seq_base, SEQ_PER_W)], tok_v)
    pltpu.sync_copy(pat_hbm.at[wid], pat_v)

    pltpu.sync_copy(zeros_hbm, stage_v)
    pltpu.sync_copy(stage_v, acc_sh.at[pl.ds(seq_base, SEQ_PER_W)])

    def _gather(g, b):
        pltpu.async_copy(proj_hbm.at[tok_v.at[g]], rows_v.at[b], gsem.at[b])

    def _gather_wait(g, b):
        pltpu.make_async_copy(proj_hbm.at[tok_v.at[g]], rows_v.at[b],
                              gsem.at[b]).wait()

    def _scatter(b):
        pltpu.async_copy(rows_v.at[b], acc_sh.at[pat_v], ssem.at[b],
                         add=True)

    def _scatter_wait(b):
        pltpu.make_async_copy(rows_v.at[b], acc_sh.at[pat_v],
                              ssem.at[b]).wait()

    for b in range(NBUF):
        _gather(b, b)

    def _step(i, carry):
        base = i * NBUF
        for b in range(NBUF):
            _gather_wait(base + b, b)
            _scatter(b)
        for b in range(NBUF):
            _scatter_wait(b)
            _gather(base + NBUF + b, b)
        return carry

    lax.fori_loop(0, L // NBUF - 1, _step, 0, unroll=False)

    last = L - NBUF
    for b in range(NBUF):
        _gather_wait(last + b, b)
        _scatter(b)
    for b in range(NBUF):
        _scatter_wait(b)

    pltpu.sync_copy(acc_sh.at[pl.ds(seq_base, SEQ_PER_W)], stage_v)
    pltpu.sync_copy(stage_v, acc_hbm.at[pl.ds(seq_base, SEQ_PER_W)])


def _fin_body(acc_ref, len_ref, b_ref, out_ref):
    out_ref[...] = acc_ref[...][:, :N_CLASSES] / len_ref[...] + b_ref[...]


def _finalize(acc, length_f32, bias):
    return pl.pallas_call(
        _fin_body,
        out_shape=jax.ShapeDtypeStruct((B, N_CLASSES), jnp.float32),
    )(acc, length_f32, bias)


def kernel(data, length, table, W, b):
    wt = jnp.zeros((EMB, DP), jnp.float32).at[:, :N_CLASSES].set(W.T)
    proj = _compute_proj(jnp.swapaxes(table, 0, 1), wt)
    tokt = jnp.swapaxes(data.astype(jnp.int32), 0, 1)
    tokt = ((tokt & ~(_ROWS_BLK - 1))
            | ((tokt & (_CHUNK - 1)) << 2)
            | ((tokt & (_ROWS_BLK - 1)) >> 10))
    pat = jnp.arange(B, dtype=jnp.int32).reshape(NW, SEQ_PER_W)
    zeros = jnp.zeros((SEQ_PER_W, DP), jnp.float32)
    acc = _sc_gather_sum(proj.reshape(_VPAD, DP), tokt, pat, zeros)
    out = _finalize(acc, length.astype(jnp.float32).reshape(B, 1),
                    b.reshape(1, N_CLASSES))
    return out


# speedup vs baseline: 23.5667x; 1.0888x over previous
import functools

import jax
import jax.numpy as jnp
from jax import lax
from jax.experimental import pallas as pl
from jax.experimental.pallas import tpu as pltpu
from jax.experimental.pallas import tpu_sc as plsc

VOCAB = 100000
EMB = 64
B = 4096
L = 200
N_CLASSES = 20
DP = 32
PACK = 128 // DP
NC = 2
NS = 16
NW = NC * NS
SEQ_PER_W = B // NW
STEP = 1
N_STEPS = L // STEP
NBUF = 10


_ROWS_BLK = 8192


_CHUNK = _ROWS_BLK // PACK
_CH_BITS = _CHUNK.bit_length() - 1
_NBLK = (VOCAB + _ROWS_BLK - 1) // _ROWS_BLK
_VPAD = _NBLK * _ROWS_BLK


def _proj_body(tblt_ref, wt4_ref, out_ref):
    t = tblt_ref[...]
    acc = None
    for k in range(PACK):
        p = lax.dot_general(t[:, k * _CHUNK:(k + 1) * _CHUNK],
                            wt4_ref[...][k * EMB:(k + 1) * EMB, :],
                            (((0,), (0,)), ((), ())),
                            preferred_element_type=jnp.float32)
        acc = p if acc is None else acc + p
    out_ref[...] = acc

    @pl.when(pl.program_id(0) == 0)
    def _zero_row0():
        out_ref[0:1, 0:DP] = jnp.zeros((1, DP), jnp.float32)


def _compute_proj(tablet, wt4):
    return pl.pallas_call(
        _proj_body,
        grid=(_NBLK,),
        in_specs=[
            pl.BlockSpec((EMB, _ROWS_BLK), lambda i: (0, i)),
            pl.BlockSpec((PACK * EMB, 128), lambda i: (0, 0)),
        ],
        out_specs=pl.BlockSpec((_CHUNK, 128), lambda i: (i, 0)),
        out_shape=jax.ShapeDtypeStruct((_VPAD // PACK, 128), jnp.float32),
        compiler_params=pltpu.CompilerParams(fuse_transposed_lhs_in_matmul=True),
    )(tablet, wt4)


_sc_mesh = plsc.VectorSubcoreMesh(core_axis_name="c", subcore_axis_name="s",
                                  num_cores=NC, num_subcores=NS)


@functools.partial(
    pl.kernel,
    out_type=jax.ShapeDtypeStruct((B, 128), jnp.float32),
    mesh=_sc_mesh,
    scratch_types=[
        pltpu.VMEM((L, SEQ_PER_W), jnp.int32),
        pltpu.VMEM((SEQ_PER_W,), jnp.int32),
        pltpu.VMEM((NBUF, SEQ_PER_W, DP), jnp.float32),
        pltpu.VMEM((SEQ_PER_W, DP), jnp.float32),
        pltpu.VMEM_SHARED((B, DP), jnp.float32),
        pltpu.SemaphoreType.DMA((NBUF,)),
        pltpu.SemaphoreType.DMA((NBUF,)),
    ],
    compiler_params=pltpu.CompilerParams(use_tc_tiling_on_sc=False),
)
def _sc_gather_sum(proj_hbm, tokt_hbm, pat_hbm, zeros_hbm, acc_hbm,
                   tok_v, pat_v, rows_v, stage_v, acc_sh, gsem, ssem):
    c = lax.axis_index("c")
    s = lax.axis_index("s")
    wid = c * NS + s
    seq_base = wid * SEQ_PER_W

    pltpu.sync_copy(tokt_hbm.at[:, pl.ds(seq_base, SEQ_PER_W)], tok_v)
    pltpu.sync_copy(pat_hbm.at[wid], pat_v)

    pltpu.sync_copy(zeros_hbm, stage_v)
    pltpu.sync_copy(stage_v, acc_sh.at[pl.ds(seq_base, SEQ_PER_W)])

    def _gather(g, b):
        pltpu.async_copy(proj_hbm.at[tok_v.at[g]], rows_v.at[b], gsem.at[b])

    def _gather_wait(g, b):
        pltpu.make_async_copy(proj_hbm.at[tok_v.at[g]], rows_v.at[b],
                              gsem.at[b]).wait()

    def _scatter(b):
        pltpu.async_copy(rows_v.at[b], acc_sh.at[pat_v], ssem.at[b],
                         add=True)

    def _scatter_wait(b):
        pltpu.make_async_copy(rows_v.at[b], acc_sh.at[pat_v],
                              ssem.at[b]).wait()

    for b in range(NBUF):
        _gather(b, b)

    def _step(i, carry):
        base = i * NBUF
        for b in range(NBUF):
            _gather_wait(base + b, b)
            _scatter(b)
        for b in range(NBUF):
            _scatter_wait(b)
            _gather(base + NBUF + b, b)
        return carry

    lax.fori_loop(0, N_STEPS // NBUF - 1, _step, 0, unroll=False)

    last = N_STEPS - NBUF
    for b in range(NBUF):
        _gather_wait(last + b, b)
        _scatter(b)
    for b in range(NBUF):
        _scatter_wait(b)

    pltpu.sync_copy(acc_sh.at[pl.ds(seq_base, SEQ_PER_W)], stage_v)
    pltpu.sync_copy(stage_v,
                    acc_hbm.at[pl.ds(seq_base, SEQ_PER_W), pl.ds(0, DP)])


def _fin_body(acc_ref, len_ref, b_ref, out_ref):
    out_ref[...] = acc_ref[:, :N_CLASSES] / len_ref[...] + b_ref[...]


def _finalize(acc, length_f32, bias):
    return pl.pallas_call(
        _fin_body,
        out_shape=jax.ShapeDtypeStruct((B, N_CLASSES), jnp.float32),
    )(acc, length_f32, bias)


def kernel(data, length, table, W, b):
    wtp = jnp.zeros((EMB, DP), jnp.float32).at[:, :N_CLASSES].set(W.T)
    wt4 = jnp.kron(jnp.eye(PACK, dtype=jnp.float32), wtp)
    proj = _compute_proj(jnp.swapaxes(table, 0, 1), wt4)
    tokt = jnp.swapaxes(data.astype(jnp.int32), 0, 1)
    tokt = ((tokt & ~(_ROWS_BLK - 1))
            | ((tokt & (_CHUNK - 1)) << 2)
            | ((tokt & (_ROWS_BLK - 1)) >> _CH_BITS))
    pat = jnp.arange(B, dtype=jnp.int32).reshape(NW, SEQ_PER_W)
    zeros = jnp.zeros((SEQ_PER_W, DP), jnp.float32)
    acc = _sc_gather_sum(proj.reshape(_VPAD, DP), tokt, pat, zeros)
    out = _finalize(acc, length.astype(jnp.float32).reshape(B, 1),
                    b.reshape(1, N_CLASSES))
    return out
